# baseline (speedup 1.0000x reference)
import functools

import jax
import jax.numpy as jnp
from jax import lax
from jax.experimental import pallas as pl
from jax.experimental.pallas import tpu as pltpu
from jax.experimental.pallas import tpu_sc as plsc

N_NODES = 10000
D_FEAT = 128
N_EDGES = 320000

NC = 2
NS = 16
L = 16
NW = NC * NS
E_W = N_EDGES // NW
B = 80
CH = E_W // B
G = B // L
K = 8


def _tile_body(xu_hbm, xm_hbm, uidx_hbm, midx_hbm, out_hbm, xu_sp,
               iu0, im0, iu1, im1, u0, m0, u1, m1, ob0, ob1,
               semi0, semi1, semr0, semr1, semo0, semo1):
    sid = lax.axis_index("s")
    wid = sid * NC + lax.axis_index("c")
    base = wid * E_W


    ibufs = ((iu0, im0, semi0), (iu1, im1, semi1))
    rbufs = ((u0, m0, semr0), (u1, m1, semr1))
    obufs = ((ob0, semo0), (ob1, semo1))
    last = CH - 1

    def start_idx(c, b):
        iu, im, sem = ibufs[b]
        pltpu.async_copy(uidx_hbm.at[pl.ds(base + c * B, B)], iu, sem)
        pltpu.async_copy(midx_hbm.at[pl.ds(base + c * B, B)], im, sem)

    def wait_idx(b):
        iu, im, sem = ibufs[b]
        pltpu.make_async_copy(uidx_hbm.at[pl.ds(base, B)], iu, sem).wait()
        pltpu.make_async_copy(midx_hbm.at[pl.ds(base, B)], im, sem).wait()

    def start_rows(b):
        iu, im, _ = ibufs[b]
        ub, mb, sem = rbufs[b]
        pltpu.async_copy(xu_hbm.at[iu], ub, sem)
        pltpu.async_copy(xm_hbm.at[im], mb, sem)

    def wait_rows(b):
        iu, im, _ = ibufs[b]
        ub, mb, sem = rbufs[b]
        pltpu.make_async_copy(xu_hbm.at[iu], ub, sem).wait()
        pltpu.make_async_copy(xm_hbm.at[im], mb, sem).wait()

    def start_out(c, b):
        ob, sem = obufs[b]
        pltpu.async_copy(ob, out_hbm.at[pl.ds(base + c * B, B)], sem)

    def wait_out(b):
        ob, sem = obufs[b]
        pltpu.make_async_copy(ob, out_hbm.at[pl.ds(base, B)], sem).wait()

    def compute(b):
        ub, mb, _ = rbufs[b]
        ob, _ = obufs[b]
        for g in range(G):
            rows = jnp.arange(L, dtype=jnp.int32) + g * L
            zero = jnp.zeros((L,), jnp.float32)
            cols0 = jnp.arange(L, dtype=jnp.int32)

            def d_body(_, carry):
                cols, *accs = carry
                new_accs = []
                for k in range(K):
                    col = ((cols + k) if k else cols) & (D_FEAT - 1)
                    uv = plsc.load_gather(ub, [rows, col])
                    mv = plsc.load_gather(mb, [rows, col])
                    new_accs.append(accs[k] + uv * mv)
                return (cols + K, *new_accs)

            res = lax.fori_loop(0, D_FEAT // K, d_body,
                                (cols0,) + (zero,) * K)
            accs = list(res[1:])
            while len(accs) > 1:
                accs = [a + b_ for a, b_ in zip(accs[::2], accs[1::2])]
            ob[pl.ds(g * L, L)] = accs[0]

    def step(c, b):
        wait_rows(b)
        wait_idx(1 - b)
        start_rows(1 - b)
        start_idx(jnp.minimum(c + 2, last), b)

        @pl.when(c >= 2)
        def _():
            wait_out(b)

        compute(b)
        start_out(c, b)

    start_idx(0, 0)
    start_idx(1, 1)
    wait_idx(0)
    start_rows(0)

    def pair_body(j, carry):
        step(2 * j, 0)
        step(2 * j + 1, 1)
        return carry

    lax.fori_loop(0, (CH - 1) // 2, pair_body, 0)

    wait_rows(0)
    wait_idx(1)
    wait_out(0)
    compute(0)
    start_out(last, 0)
    wait_out(0)
    wait_out(1)


@functools.partial(
    pl.kernel,
    mesh=plsc.VectorSubcoreMesh(core_axis_name="c", subcore_axis_name="s"),
    out_type=jax.ShapeDtypeStruct((N_EDGES,), jnp.float32),
    compiler_params=pltpu.CompilerParams(needs_layout_passes=False),
    scratch_types=[
        pltpu.VMEM_SHARED((N_NODES, D_FEAT), jnp.float32),
        pltpu.VMEM((B,), jnp.int32),
        pltpu.VMEM((B,), jnp.int32),
        pltpu.VMEM((B,), jnp.int32),
        pltpu.VMEM((B,), jnp.int32),
        pltpu.VMEM((B, D_FEAT), jnp.float32),
        pltpu.VMEM((B, D_FEAT), jnp.float32),
        pltpu.VMEM((B, D_FEAT), jnp.float32),
        pltpu.VMEM((B, D_FEAT), jnp.float32),
        pltpu.VMEM((B,), jnp.float32),
        pltpu.VMEM((B,), jnp.float32),
        pltpu.SemaphoreType.DMA,
        pltpu.SemaphoreType.DMA,
        pltpu.SemaphoreType.DMA,
        pltpu.SemaphoreType.DMA,
        pltpu.SemaphoreType.DMA,
        pltpu.SemaphoreType.DMA,
    ],
)
def _edge_dot_sc(xu_hbm, xm_hbm, uidx_hbm, midx_hbm, out_hbm, xu_sp,
                 iu0, im0, iu1, im1, u0, m0, u1, m1, ob0, ob1,
                 semi0, semi1, semr0, semr1, semo0, semo1):
    _tile_body(xu_hbm, xm_hbm, uidx_hbm, midx_hbm, out_hbm, xu_sp,
               iu0, im0, iu1, im1, u0, m0, u1, m1, ob0, ob1,
               semi0, semi1, semr0, semr1, semo0, semo1)


def kernel(x_user, x_movie, edge_label_index):
    idx = edge_label_index.astype(jnp.int32)
    return _edge_dot_sc(x_user, x_movie, idx[0], idx[1])


# speedup vs baseline: 1.2793x; 1.2793x over previous
import functools

import jax
import jax.numpy as jnp
from jax import lax
from jax.experimental import pallas as pl
from jax.experimental.pallas import tpu as pltpu
from jax.experimental.pallas import tpu_sc as plsc

N_NODES = 10000
D_FEAT = 128
N_EDGES = 320000

NC = 2
NS = 16
L = 16
NW = NC * NS
E_W = N_EDGES // NW
B = 80
CH = E_W // B
G = B // L
K = 8


def _tile_body(xu_hbm, xm_hbm, uidx_hbm, midx_hbm, out_hbm, xu_sp,
               iu0, im0, iu1, im1, u0, m0, u1, m1, ob0, ob1,
               semi0, semi1, semr0, semr1, semo0, semo1, semu0, semu1):
    sid = lax.axis_index("s")
    wid = sid * NC + lax.axis_index("c")
    base = wid * E_W


    R_T = 624
    pltpu.sync_copy(xu_hbm.at[pl.ds(sid * R_T, R_T)],
                    xu_sp.at[pl.ds(sid * R_T, R_T)])
    rem = NS * R_T

    @pl.when(sid == NS - 1)
    def _():
        pltpu.sync_copy(xu_hbm.at[pl.ds(rem, N_NODES - rem)],
                        xu_sp.at[pl.ds(rem, N_NODES - rem)])

    plsc.subcore_barrier()

    ibufs = ((iu0, im0, semi0), (iu1, im1, semi1))
    rbufs = ((u0, m0, semr0, semu0), (u1, m1, semr1, semu1))
    obufs = ((ob0, semo0), (ob1, semo1))
    last = CH - 1

    def start_idx(c, b):
        iu, im, sem = ibufs[b]
        pltpu.async_copy(uidx_hbm.at[pl.ds(base + c * B, B)], iu, sem)
        pltpu.async_copy(midx_hbm.at[pl.ds(base + c * B, B)], im, sem)

    def wait_idx(b):
        iu, im, sem = ibufs[b]
        pltpu.make_async_copy(uidx_hbm.at[pl.ds(base, B)], iu, sem).wait()
        pltpu.make_async_copy(midx_hbm.at[pl.ds(base, B)], im, sem).wait()

    def start_rows(b):
        iu, im, _ = ibufs[b]
        ub, mb, sem, semu = rbufs[b]
        pltpu.async_copy(xu_sp.at[iu], ub, semu)
        pltpu.async_copy(xm_hbm.at[im], mb, sem)

    def wait_rows(b):
        iu, im, _ = ibufs[b]
        ub, mb, sem, semu = rbufs[b]
        pltpu.make_async_copy(xu_sp.at[iu], ub, semu).wait()
        pltpu.make_async_copy(xm_hbm.at[im], mb, sem).wait()

    def start_out(c, b):
        ob, sem = obufs[b]
        pltpu.async_copy(ob, out_hbm.at[pl.ds(base + c * B, B)], sem)

    def wait_out(b):
        ob, sem = obufs[b]
        pltpu.make_async_copy(ob, out_hbm.at[pl.ds(base, B)], sem).wait()

    def compute(b):
        ub, mb = rbufs[b][0], rbufs[b][1]
        ob, _ = obufs[b]
        for g in range(G):
            rows = jnp.arange(L, dtype=jnp.int32) + g * L
            zero = jnp.zeros((L,), jnp.float32)
            cols0 = jnp.arange(L, dtype=jnp.int32)

            def d_body(_, carry):
                cols, *accs = carry
                new_accs = []
                for k in range(K):
                    col = ((cols + k) if k else cols) & (D_FEAT - 1)
                    uv = plsc.load_gather(ub, [rows, col])
                    mv = plsc.load_gather(mb, [rows, col])
                    new_accs.append(accs[k] + uv * mv)
                return (cols + K, *new_accs)

            res = lax.fori_loop(0, D_FEAT // K, d_body,
                                (cols0,) + (zero,) * K)
            accs = list(res[1:])
            while len(accs) > 1:
                accs = [a + b_ for a, b_ in zip(accs[::2], accs[1::2])]
            ob[pl.ds(g * L, L)] = accs[0]

    def step(c, b):
        wait_rows(b)
        wait_idx(1 - b)
        start_rows(1 - b)
        start_idx(jnp.minimum(c + 2, last), b)

        @pl.when(c >= 2)
        def _():
            wait_out(b)

        compute(b)
        start_out(c, b)

    start_idx(0, 0)
    start_idx(1, 1)
    wait_idx(0)
    start_rows(0)

    def pair_body(j, carry):
        step(2 * j, 0)
        step(2 * j + 1, 1)
        return carry

    lax.fori_loop(0, (CH - 1) // 2, pair_body, 0)

    wait_rows(0)
    wait_idx(1)
    wait_out(0)
    compute(0)
    start_out(last, 0)
    wait_out(0)
    wait_out(1)


@functools.partial(
    pl.kernel,
    mesh=plsc.VectorSubcoreMesh(core_axis_name="c", subcore_axis_name="s"),
    out_type=jax.ShapeDtypeStruct((N_EDGES,), jnp.float32),
    compiler_params=pltpu.CompilerParams(needs_layout_passes=False),
    scratch_types=[
        pltpu.VMEM_SHARED((N_NODES, D_FEAT), jnp.float32),
        pltpu.VMEM((B,), jnp.int32),
        pltpu.VMEM((B,), jnp.int32),
        pltpu.VMEM((B,), jnp.int32),
        pltpu.VMEM((B,), jnp.int32),
        pltpu.VMEM((B, D_FEAT), jnp.float32),
        pltpu.VMEM((B, D_FEAT), jnp.float32),
        pltpu.VMEM((B, D_FEAT), jnp.float32),
        pltpu.VMEM((B, D_FEAT), jnp.float32),
        pltpu.VMEM((B,), jnp.float32),
        pltpu.VMEM((B,), jnp.float32),
        pltpu.SemaphoreType.DMA,
        pltpu.SemaphoreType.DMA,
        pltpu.SemaphoreType.DMA,
        pltpu.SemaphoreType.DMA,
        pltpu.SemaphoreType.DMA,
        pltpu.SemaphoreType.DMA,
        pltpu.SemaphoreType.DMA,
        pltpu.SemaphoreType.DMA,
    ],
)
def _edge_dot_sc(xu_hbm, xm_hbm, uidx_hbm, midx_hbm, out_hbm, xu_sp,
                 iu0, im0, iu1, im1, u0, m0, u1, m1, ob0, ob1,
                 semi0, semi1, semr0, semr1, semo0, semo1, semu0, semu1):
    _tile_body(xu_hbm, xm_hbm, uidx_hbm, midx_hbm, out_hbm, xu_sp,
               iu0, im0, iu1, im1, u0, m0, u1, m1, ob0, ob1,
               semi0, semi1, semr0, semr1, semo0, semo1, semu0, semu1)


def kernel(x_user, x_movie, edge_label_index):
    idx = edge_label_index.astype(jnp.int32)
    return _edge_dot_sc(x_user, x_movie, idx[0], idx[1])
